# baseline (speedup 1.0000x reference)
import functools

import jax
import jax.numpy as jnp
from jax import lax
from jax.experimental import pallas as pl
from jax.experimental.pallas import tpu as pltpu
from jax.experimental.pallas import tpu_sc as plsc

R = 16384
C = 1000
NC, NS, L = 2, 16, 16
NW = NC * NS

SC_NT = 8
SC_ROWS = NW * SC_NT * L
TC_ROWS = R - SC_ROWS
SC_TILES = SC_ROWS // L

NPAIR = SC_NT // 2
TW = L * C
A = 4
SEG = C // A
Q = 5

BR = 256

_mesh = plsc.VectorSubcoreMesh(core_axis_name="c", subcore_axis_name="s")


@functools.partial(
    pl.kernel,
    out_type=jax.ShapeDtypeStruct((SC_TILES, TW), jnp.float32),
    mesh=_mesh,
    scratch_types=[
        pltpu.VMEM((TW,), jnp.float32),
        pltpu.VMEM((TW,), jnp.float32),
        pltpu.VMEM((TW,), jnp.float32),
        pltpu.VMEM((TW,), jnp.float32),
        pltpu.SemaphoreType.DMA,
        pltpu.SemaphoreType.DMA,
        pltpu.SemaphoreType.DMA,
        pltpu.SemaphoreType.DMA,
    ],
    compiler_params=pltpu.CompilerParams(needs_layout_passes=False),
)
def _onehot_sc(x_hbm, out_hbm, xa, xb, oa, ob, sxa, sxb, soa, sob):
    wid = lax.axis_index("s") * NC + lax.axis_index("c")
    lane_base = lax.iota(jnp.int32, L) * C
    zeros = jnp.zeros((L,), jnp.float32)
    ones = jnp.ones((L,), jnp.float32)
    minf = jnp.full((L,), -jnp.inf, jnp.float32)
    wtile = wid * SC_NT

    def start_x(tid, xv, sx):
        pltpu.async_copy(x_hbm.at[tid], xv, sx)

    def wait_x(tid, xv, sx):
        pltpu.make_async_copy(x_hbm.at[tid], xv, sx).wait()

    def start_out(tid, ov, so):
        pltpu.async_copy(ov, out_hbm.at[tid], so)

    def wait_out(tid, ov, so):
        pltpu.make_async_copy(ov, out_hbm.at[tid], so).wait()

    def zero_body(i, _):
        oa[pl.ds(i * L, L)] = zeros
        ob[pl.ds(i * L, L)] = zeros
        return 0

    lax.fori_loop(0, TW // L, zero_body, 0)

    start_x(wtile, xa, sxa)
    start_x(wtile + 1, xb, sxb)

    def half(i, t, xv, ov, sx, so, prev_hot):
        tid = wtile + t
        wait_x(tid, xv, sx)

        def amax_body(jj, carry):
            mvs, mos, off = carry
            mvs, mos = list(mvs), list(mos)
            for q in range(Q):
                for a in range(A):
                    o = off + (a * SEG + q)
                    col = plsc.load_gather(xv, [o])
                    pred = col > mvs[a]
                    mos[a] = jnp.where(pred, o, mos[a])
                    mvs[a] = jnp.maximum(mvs[a], col)
            return tuple(mvs), tuple(mos), off + Q

        init_mos = tuple(lane_base + a * SEG for a in range(A))
        mvs, mos, _ = lax.fori_loop(0, SEG // Q, amax_body,
                                    ((minf,) * A, init_mos, lane_base))
        mv, mo = mvs[0], mos[0]
        for a in range(1, A):
            pred = mvs[a] > mv
            mo = jnp.where(pred, mos[a], mo)
            mv = jnp.where(pred, mvs[a], mv)

        @pl.when(i > 0)
        def _wait_out():
            wait_out(tid, ov, so)

        plsc.store_scatter(ov, [prev_hot], zeros)
        plsc.store_scatter(ov, [mo], ones)
        start_out(tid, ov, so)

        @pl.when(i < NPAIR - 1)
        def _next_x():
            start_x(tid + 2, xv, sx)

        return mo

    def pair_body(i, carry):
        pa, pb = carry
        pa = half(i, 2 * i, xa, oa, sxa, soa, pa)
        pb = half(i, 2 * i + 1, xb, ob, sxb, sob, pb)
        return (pa, pb)

    lax.fori_loop(0, NPAIR, pair_body, (lane_base, lane_base))

    wait_out(0, oa, soa)
    wait_out(0, ob, sob)


def _tc_body(x_ref, o_ref):
    xb = x_ref[...]
    iota = lax.broadcasted_iota(jnp.int32, (BR, C), 1)
    m = jnp.max(xb, axis=1, keepdims=True)
    cand = jnp.where(xb == m, iota, C)
    idx = jnp.min(cand, axis=1, keepdims=True)
    o_ref[...] = (iota == idx).astype(jnp.float32)


_onehot_tc = pl.pallas_call(
    _tc_body,
    grid=(TC_ROWS // BR,),
    in_specs=[pl.BlockSpec((BR, C), lambda i: (i, 0))],
    out_specs=pl.BlockSpec((BR, C), lambda i: (i, 0)),
    out_shape=jax.ShapeDtypeStruct((TC_ROWS, C), jnp.float32),
)


def kernel(x):
    tc_out = _onehot_tc(x[:TC_ROWS])
    sc_out = _onehot_sc(x[TC_ROWS:].reshape(SC_TILES, TW))
    return jnp.concatenate([tc_out, sc_out.reshape(SC_ROWS, C)], axis=0)


# speedup vs baseline: 1.0982x; 1.0982x over previous
import functools

import jax
import jax.numpy as jnp
from jax import lax
from jax.experimental import pallas as pl
from jax.experimental.pallas import tpu as pltpu
from jax.experimental.pallas import tpu_sc as plsc

R = 16384
C = 1000
NC, NS, L = 2, 16, 16
NW = NC * NS

SC_NT = 8
SC_ROWS = NW * SC_NT * L
TC_ROWS = R - SC_ROWS
SC_TILES = SC_ROWS // L

NPAIR = SC_NT // 2
TW = L * C
A = 4
SEG = C // A
Q = 5

BR = 256

_mesh = plsc.VectorSubcoreMesh(core_axis_name="c", subcore_axis_name="s")


@functools.partial(
    pl.kernel,
    out_type=jax.ShapeDtypeStruct((SC_TILES, TW), jnp.float32),
    mesh=_mesh,
    scratch_types=[
        pltpu.VMEM((TW,), jnp.float32),
        pltpu.VMEM((TW,), jnp.float32),
        pltpu.VMEM((TW,), jnp.float32),
        pltpu.VMEM((TW,), jnp.float32),
        pltpu.SemaphoreType.DMA,
        pltpu.SemaphoreType.DMA,
        pltpu.SemaphoreType.DMA,
        pltpu.SemaphoreType.DMA,
    ],
    compiler_params=pltpu.CompilerParams(needs_layout_passes=False),
)
def _onehot_sc(x_hbm, out_hbm, xa, xb, oa, ob, sxa, sxb, soa, sob):
    wid = lax.axis_index("s") * NC + lax.axis_index("c")
    lane_base = lax.iota(jnp.int32, L) * C
    zeros = jnp.zeros((L,), jnp.float32)
    ones = jnp.ones((L,), jnp.float32)
    minf = jnp.full((L,), -jnp.inf, jnp.float32)
    wtile = wid * SC_NT

    def start_x(tid, xv, sx):
        pltpu.async_copy(x_hbm.at[tid], xv, sx)

    def wait_x(tid, xv, sx):
        pltpu.make_async_copy(x_hbm.at[tid], xv, sx).wait()

    def start_out(tid, ov, so):
        pltpu.async_copy(ov, out_hbm.at[tid], so)

    def wait_out(tid, ov, so):
        pltpu.make_async_copy(ov, out_hbm.at[tid], so).wait()

    def zero_body(i, _):
        oa[pl.ds(i * L, L)] = zeros
        ob[pl.ds(i * L, L)] = zeros
        return 0

    lax.fori_loop(0, TW // L, zero_body, 0)

    start_x(wtile, xa, sxa)
    start_x(wtile + 1, xb, sxb)

    def half(i, t, xv, ov, sx, so, prev_hot):
        tid = wtile + t
        wait_x(tid, xv, sx)

        def amax_body(jj, carry):
            mvs, mos, off = carry
            mvs, mos = list(mvs), list(mos)
            for q in range(Q):
                for a in range(A):
                    o = off + (a * SEG + q)
                    col = plsc.load_gather(xv, [o])
                    pred = col > mvs[a]
                    mos[a] = jnp.where(pred, o, mos[a])
                    mvs[a] = jnp.maximum(mvs[a], col)
            return tuple(mvs), tuple(mos), off + Q

        init_mos = tuple(lane_base + a * SEG for a in range(A))
        mvs, mos, _ = lax.fori_loop(0, SEG // Q, amax_body,
                                    ((minf,) * A, init_mos, lane_base))
        mv, mo = mvs[0], mos[0]
        for a in range(1, A):
            pred = mvs[a] > mv
            mo = jnp.where(pred, mos[a], mo)
            mv = jnp.where(pred, mvs[a], mv)

        @pl.when(i > 0)
        def _wait_out():
            wait_out(tid, ov, so)

        plsc.store_scatter(ov, [prev_hot], zeros)
        plsc.store_scatter(ov, [mo], ones)
        start_out(tid, ov, so)

        @pl.when(i < NPAIR - 1)
        def _next_x():
            start_x(tid + 2, xv, sx)

        return mo

    def pair_body(i, carry):
        pa, pb = carry
        pa = half(i, 2 * i, xa, oa, sxa, soa, pa)
        pb = half(i, 2 * i + 1, xb, ob, sxb, sob, pb)
        return (pa, pb)

    lax.fori_loop(0, NPAIR, pair_body, (lane_base, lane_base))

    wait_out(0, oa, soa)
    wait_out(0, ob, sob)


def _tc_body(x_ref, o_ref):
    xb = x_ref[...]
    iota = lax.broadcasted_iota(jnp.int32, (BR, C), 1)
    m = jnp.max(xb, axis=1, keepdims=True)
    cand = jnp.where(xb == m, iota, C)
    idx = jnp.min(cand, axis=1, keepdims=True)
    o_ref[...] = (iota == idx).astype(jnp.float32)


_onehot_tc = pl.pallas_call(
    _tc_body,
    grid=(TC_ROWS // BR,),
    in_specs=[pl.BlockSpec((BR, C), lambda i: (i, 0))],
    out_specs=pl.BlockSpec((BR, C), lambda i: (i, 0)),
    out_shape=jax.ShapeDtypeStruct((R, C), jnp.float32),
)


def kernel(x):
    tc_out = _onehot_tc(x[:TC_ROWS])
    sc_out = _onehot_sc(x[TC_ROWS:].reshape(SC_TILES, TW))
    return lax.dynamic_update_slice(tc_out, sc_out.reshape(SC_ROWS, C),
                                    (TC_ROWS, 0))


# speedup vs baseline: 1.1587x; 1.0551x over previous
import functools

import jax
import jax.numpy as jnp
from jax import lax
from jax.experimental import pallas as pl
from jax.experimental.pallas import tpu as pltpu
from jax.experimental.pallas import tpu_sc as plsc

R = 16384
C = 1000
NC, NS, L = 2, 16, 16
NW = NC * NS

SC_NT = 8
SC_ROWS = NW * SC_NT * L
TC_ROWS = R - SC_ROWS
SC_TILES = SC_ROWS // L

NPAIR = SC_NT // 2
TW = L * C
A = 4
SEG = C // A
Q = 5

BR = 512

_mesh = plsc.VectorSubcoreMesh(core_axis_name="c", subcore_axis_name="s")


@functools.partial(
    pl.kernel,
    out_type=jax.ShapeDtypeStruct((SC_TILES, TW), jnp.float32),
    mesh=_mesh,
    scratch_types=[
        pltpu.VMEM((TW,), jnp.float32),
        pltpu.VMEM((TW,), jnp.float32),
        pltpu.VMEM((TW,), jnp.float32),
        pltpu.VMEM((TW,), jnp.float32),
        pltpu.SemaphoreType.DMA,
        pltpu.SemaphoreType.DMA,
        pltpu.SemaphoreType.DMA,
        pltpu.SemaphoreType.DMA,
    ],
    compiler_params=pltpu.CompilerParams(needs_layout_passes=False),
)
def _onehot_sc(x_hbm, out_hbm, xa, xb, oa, ob, sxa, sxb, soa, sob):
    wid = lax.axis_index("s") * NC + lax.axis_index("c")
    lane_base = lax.iota(jnp.int32, L) * C
    zeros = jnp.zeros((L,), jnp.float32)
    ones = jnp.ones((L,), jnp.float32)
    minf = jnp.full((L,), -jnp.inf, jnp.float32)
    wtile = wid * SC_NT

    def start_x(tid, xv, sx):
        pltpu.async_copy(x_hbm.at[tid], xv, sx)

    def wait_x(tid, xv, sx):
        pltpu.make_async_copy(x_hbm.at[tid], xv, sx).wait()

    def start_out(tid, ov, so):
        pltpu.async_copy(ov, out_hbm.at[tid], so)

    def wait_out(tid, ov, so):
        pltpu.make_async_copy(ov, out_hbm.at[tid], so).wait()

    def zero_body(i, _):
        oa[pl.ds(i * L, L)] = zeros
        ob[pl.ds(i * L, L)] = zeros
        return 0

    lax.fori_loop(0, TW // L, zero_body, 0)

    start_x(wtile, xa, sxa)
    start_x(wtile + 1, xb, sxb)

    def half(i, t, xv, ov, sx, so, prev_hot):
        tid = wtile + t
        wait_x(tid, xv, sx)

        def amax_body(jj, carry):
            mvs, mos, off = carry
            mvs, mos = list(mvs), list(mos)
            for q in range(Q):
                for a in range(A):
                    o = off + (a * SEG + q)
                    col = plsc.load_gather(xv, [o])
                    pred = col > mvs[a]
                    mos[a] = jnp.where(pred, o, mos[a])
                    mvs[a] = jnp.maximum(mvs[a], col)
            return tuple(mvs), tuple(mos), off + Q

        init_mos = tuple(lane_base + a * SEG for a in range(A))
        mvs, mos, _ = lax.fori_loop(0, SEG // Q, amax_body,
                                    ((minf,) * A, init_mos, lane_base))
        mv, mo = mvs[0], mos[0]
        for a in range(1, A):
            pred = mvs[a] > mv
            mo = jnp.where(pred, mos[a], mo)
            mv = jnp.where(pred, mvs[a], mv)

        @pl.when(i > 0)
        def _wait_out():
            wait_out(tid, ov, so)

        plsc.store_scatter(ov, [prev_hot], zeros)
        plsc.store_scatter(ov, [mo], ones)
        start_out(tid, ov, so)

        @pl.when(i < NPAIR - 1)
        def _next_x():
            start_x(tid + 2, xv, sx)

        return mo

    def pair_body(i, carry):
        pa, pb = carry
        pa = half(i, 2 * i, xa, oa, sxa, soa, pa)
        pb = half(i, 2 * i + 1, xb, ob, sxb, sob, pb)
        return (pa, pb)

    lax.fori_loop(0, NPAIR, pair_body, (lane_base, lane_base))

    wait_out(0, oa, soa)
    wait_out(0, ob, sob)


def _tc_body(x_ref, o_ref):
    xb = x_ref[...]
    iota = lax.broadcasted_iota(jnp.int32, (BR, C), 1)
    m = jnp.max(xb, axis=1, keepdims=True)
    cand = jnp.where(xb == m, iota, C)
    idx = jnp.min(cand, axis=1, keepdims=True)
    o_ref[...] = (iota == idx).astype(jnp.float32)


_onehot_tc = pl.pallas_call(
    _tc_body,
    grid=(TC_ROWS // BR,),
    in_specs=[pl.BlockSpec((BR, C), lambda i: (i, 0))],
    out_specs=pl.BlockSpec((BR, C), lambda i: (i, 0)),
    out_shape=jax.ShapeDtypeStruct((R, C), jnp.float32),
)


def kernel(x):
    tc_out = _onehot_tc(x[:TC_ROWS])
    sc_out = _onehot_sc(x[TC_ROWS:].reshape(SC_TILES, TW))
    return lax.dynamic_update_slice(tc_out, sc_out.reshape(SC_ROWS, C),
                                    (TC_ROWS, 0))


# speedup vs baseline: 1.2545x; 1.0827x over previous
import functools

import jax
import jax.numpy as jnp
from jax import lax
from jax.experimental import pallas as pl
from jax.experimental.pallas import tpu as pltpu
from jax.experimental.pallas import tpu_sc as plsc

R = 16384
C = 1000
NC, NS, L = 2, 16, 16
NW = NC * NS

SC_NT = 4
SC_ROWS = NW * SC_NT * L
TC_ROWS = R - SC_ROWS
SC_TILES = SC_ROWS // L

NPAIR = SC_NT // 2
TW = L * C
A = 4
SEG = C // A
Q = 5

BR = 512

_mesh = plsc.VectorSubcoreMesh(core_axis_name="c", subcore_axis_name="s")


@functools.partial(
    pl.kernel,
    out_type=jax.ShapeDtypeStruct((SC_TILES, TW), jnp.float32),
    mesh=_mesh,
    scratch_types=[
        pltpu.VMEM((TW,), jnp.float32),
        pltpu.VMEM((TW,), jnp.float32),
        pltpu.VMEM((TW,), jnp.float32),
        pltpu.VMEM((TW,), jnp.float32),
        pltpu.SemaphoreType.DMA,
        pltpu.SemaphoreType.DMA,
        pltpu.SemaphoreType.DMA,
        pltpu.SemaphoreType.DMA,
    ],
    compiler_params=pltpu.CompilerParams(needs_layout_passes=False),
)
def _onehot_sc(x_hbm, out_hbm, xa, xb, oa, ob, sxa, sxb, soa, sob):
    wid = lax.axis_index("s") * NC + lax.axis_index("c")
    lane_base = lax.iota(jnp.int32, L) * C
    zeros = jnp.zeros((L,), jnp.float32)
    ones = jnp.ones((L,), jnp.float32)
    minf = jnp.full((L,), -jnp.inf, jnp.float32)
    wtile = wid * SC_NT

    def start_x(tid, xv, sx):
        pltpu.async_copy(x_hbm.at[tid], xv, sx)

    def wait_x(tid, xv, sx):
        pltpu.make_async_copy(x_hbm.at[tid], xv, sx).wait()

    def start_out(tid, ov, so):
        pltpu.async_copy(ov, out_hbm.at[tid], so)

    def wait_out(tid, ov, so):
        pltpu.make_async_copy(ov, out_hbm.at[tid], so).wait()

    def zero_body(i, _):
        oa[pl.ds(i * L, L)] = zeros
        ob[pl.ds(i * L, L)] = zeros
        return 0

    lax.fori_loop(0, TW // L, zero_body, 0)

    start_x(wtile, xa, sxa)
    start_x(wtile + 1, xb, sxb)

    def half(i, t, xv, ov, sx, so, prev_hot):
        tid = wtile + t
        wait_x(tid, xv, sx)

        def amax_body(jj, carry):
            mvs, mos, off = carry
            mvs, mos = list(mvs), list(mos)
            for q in range(Q):
                for a in range(A):
                    o = off + (a * SEG + q)
                    col = plsc.load_gather(xv, [o])
                    pred = col > mvs[a]
                    mos[a] = jnp.where(pred, o, mos[a])
                    mvs[a] = jnp.maximum(mvs[a], col)
            return tuple(mvs), tuple(mos), off + Q

        init_mos = tuple(lane_base + a * SEG for a in range(A))
        mvs, mos, _ = lax.fori_loop(0, SEG // Q, amax_body,
                                    ((minf,) * A, init_mos, lane_base))
        mv, mo = mvs[0], mos[0]
        for a in range(1, A):
            pred = mvs[a] > mv
            mo = jnp.where(pred, mos[a], mo)
            mv = jnp.where(pred, mvs[a], mv)

        @pl.when(i > 0)
        def _wait_out():
            wait_out(tid, ov, so)

        plsc.store_scatter(ov, [prev_hot], zeros)
        plsc.store_scatter(ov, [mo], ones)
        start_out(tid, ov, so)

        @pl.when(i < NPAIR - 1)
        def _next_x():
            start_x(tid + 2, xv, sx)

        return mo

    def pair_body(i, carry):
        pa, pb = carry
        pa = half(i, 2 * i, xa, oa, sxa, soa, pa)
        pb = half(i, 2 * i + 1, xb, ob, sxb, sob, pb)
        return (pa, pb)

    lax.fori_loop(0, NPAIR, pair_body, (lane_base, lane_base))

    wait_out(0, oa, soa)
    wait_out(0, ob, sob)


def _tc_body(x_ref, o_ref):
    xb = x_ref[...]
    iota = lax.broadcasted_iota(jnp.int32, (BR, C), 1)
    m = jnp.max(xb, axis=1, keepdims=True)
    cand = jnp.where(xb == m, iota, C)
    idx = jnp.min(cand, axis=1, keepdims=True)
    o_ref[...] = (iota == idx).astype(jnp.float32)


_onehot_tc = pl.pallas_call(
    _tc_body,
    grid=(TC_ROWS // BR,),
    in_specs=[pl.BlockSpec((BR, C), lambda i: (i, 0))],
    out_specs=pl.BlockSpec((BR, C), lambda i: (i, 0)),
    out_shape=jax.ShapeDtypeStruct((R, C), jnp.float32),
)


def kernel(x):
    tc_out = _onehot_tc(x[:TC_ROWS])
    sc_out = _onehot_sc(x[TC_ROWS:].reshape(SC_TILES, TW))
    return lax.dynamic_update_slice(tc_out, sc_out.reshape(SC_ROWS, C),
                                    (TC_ROWS, 0))
